# baseline (speedup 1.0000x reference)
import functools

import jax
import jax.numpy as jnp
from jax import lax
from jax.experimental import pallas as pl
from jax.experimental.pallas import tpu as pltpu
from jax.experimental.pallas import tpu_sc as plsc

ENT_N = 1000000
REL_N = 1000000
D = 64
BS = 4096
BSEQ = 8192
MARGIN_C = 1.0
REG_C = 1e-05

NW = 32
E_IDX = 2 * BSEQ
E_PER = E_IDX // NW
R_PER = BSEQ // NW
GCHUNK = 128

@functools.cache
def _sc_gather_fn():
    mesh = plsc.VectorSubcoreMesh(core_axis_name="c", subcore_axis_name="s")

    @functools.partial(
        pl.kernel,
        mesh=mesh,
        out_type=[
            jax.ShapeDtypeStruct((E_IDX, D), jnp.float32),
            jax.ShapeDtypeStruct((E_IDX, D), jnp.float32),
            jax.ShapeDtypeStruct((BSEQ, D), jnp.float32),
            jax.ShapeDtypeStruct((BSEQ, D), jnp.float32),
        ],
        scratch_types=[
            pltpu.VMEM((E_PER,), jnp.int32),
            pltpu.VMEM((R_PER,), jnp.int32),
            pltpu.VMEM((E_PER, D), jnp.float32),
            pltpu.VMEM((E_PER, D), jnp.float32),
            pltpu.VMEM((R_PER, D), jnp.float32),
            pltpu.VMEM((R_PER, D), jnp.float32),
            pltpu.SemaphoreType.DMA,
        ],
        compiler_params=pltpu.CompilerParams(use_tc_tiling_on_sc=False),
    )
    def _sc_gather(idx_e_hbm, idx_r_hbm, ent_emb, ent_tr, rel_emb, rel_tr,
                   ht_out, httr_out, r_out, rtr_out,
                   idx_e_v, idx_r_v, rows_he, rows_htr, rows_r, rows_rtr, sem):
        wid = lax.axis_index("s") * 2 + lax.axis_index("c")
        be = wid * E_PER
        br = wid * R_PER
        pltpu.sync_copy(idx_e_hbm.at[pl.ds(be, E_PER)], idx_e_v)
        pltpu.sync_copy(idx_r_hbm.at[pl.ds(br, R_PER)], idx_r_v)
        copies = []
        for j in range(E_PER // GCHUNK):
            s = pl.ds(j * GCHUNK, GCHUNK)
            copies.append(pltpu.async_copy(ent_emb.at[idx_e_v.at[s]], rows_he.at[s], sem))
            copies.append(pltpu.async_copy(ent_tr.at[idx_e_v.at[s]], rows_htr.at[s], sem))
        for j in range(R_PER // GCHUNK):
            s = pl.ds(j * GCHUNK, GCHUNK)
            copies.append(pltpu.async_copy(rel_emb.at[idx_r_v.at[s]], rows_r.at[s], sem))
            copies.append(pltpu.async_copy(rel_tr.at[idx_r_v.at[s]], rows_rtr.at[s], sem))
        for c in copies:
            c.wait()
        pltpu.sync_copy(rows_he, ht_out.at[pl.ds(be, E_PER)])
        pltpu.sync_copy(rows_htr, httr_out.at[pl.ds(be, E_PER)])
        pltpu.sync_copy(rows_r, r_out.at[pl.ds(br, R_PER)])
        pltpu.sync_copy(rows_rtr, rtr_out.at[pl.ds(br, R_PER)])

    return _sc_gather


def _loss_body(ht_ref, httr_ref, r_ref, rtr_ref, out_ref):
    r_tr = rtr_ref[...]

    def transfer(e, etr):
        dot = jnp.sum(e * etr, axis=1, keepdims=True)
        e2 = e + dot * r_tr
        n = jnp.sqrt(jnp.sum(e2 * e2, axis=1, keepdims=True))
        return e2 / jnp.maximum(n, 1e-12)

    h = transfer(ht_ref[0:BSEQ, :], httr_ref[0:BSEQ, :])
    t = transfer(ht_ref[BSEQ:E_IDX, :], httr_ref[BSEQ:E_IDX, :])
    a = jnp.abs(h + r_ref[...] - t + 1e-06)
    diff = a[0:BS, :] - a[BS:BSEQ, :]
    rows = jnp.sum(diff, axis=1, keepdims=True)
    out_ref[0, 0] = jnp.sum(jnp.maximum(rows + MARGIN_C, 0.0)) * (1.0 / BS)


_loss_call = pl.pallas_call(
    _loss_body,
    out_specs=pl.BlockSpec(memory_space=pltpu.SMEM),
    out_shape=jax.ShapeDtypeStruct((1, 1), jnp.float32),
)

NCHUNK = 8000
NGRID = ENT_N // NCHUNK


def _norm_body(a_ref, b_ref, c_ref, d_ref, out_ref):
    i = pl.program_id(0)
    s_ent = jnp.sum(jnp.abs(a_ref[...])) + jnp.sum(jnp.abs(c_ref[...]))
    s_rel = jnp.sum(jnp.abs(b_ref[...])) + jnp.sum(jnp.abs(d_ref[...]))
    val = s_ent * (1.0 / ENT_N) + s_rel * (1.0 / REL_N)

    @pl.when(i == 0)
    def _():
        out_ref[0, 0] = 0.0

    out_ref[0, 0] += val


_norm_call = pl.pallas_call(
    _norm_body,
    grid=(NGRID,),
    in_specs=[pl.BlockSpec((NCHUNK, D), lambda i: (i, 0))] * 4,
    out_specs=pl.BlockSpec(memory_space=pltpu.SMEM),
    out_shape=jax.ShapeDtypeStruct((1, 1), jnp.float32),
)


def kernel(input, ent_emb, rel_emb, ent_transfer, rel_transfer):
    idx_e = jnp.concatenate([input[:, 0], input[:, 2]])
    idx_r = input[:, 1]
    ht, httr, r, rtr = _sc_gather_fn()(idx_e, idx_r, ent_emb, ent_transfer,
                                       rel_emb, rel_transfer)
    loss = _loss_call(ht, httr, r, rtr)
    norm = _norm_call(ent_emb, rel_emb, ent_transfer, rel_transfer)
    return loss[0, 0] + norm[0, 0] * REG_C
